# baseline (speedup 1.0000x reference)
import functools

import jax
import jax.numpy as jnp
from jax import lax
from jax.experimental import pallas as pl
from jax.experimental.pallas import tpu as pltpu
from jax.experimental.pallas import tpu_sc as plsc

NUM_CODEBOOKS = 4
CODEBOOK_BITS = 17
CODEBOOK_DIM = 16
HIDDEN_SIZE = 768
B, S = 4, 4096
T = B * S
V = NUM_CODEBOOKS * (1 << CODEBOOK_BITS)
RP128 = 128 // CODEBOOK_DIM

_RB = 2048
_NRB = V // RP128 // _RB


def _repack_body(in_ref, out_ref):
    for k in range(RP128):
        out_ref[:, k * CODEBOOK_DIM:(k + 1) * CODEBOOK_DIM] = in_ref[:, k, :]


def _repack_tc(table3d):
    return pl.pallas_call(
        _repack_body,
        grid=(_NRB,),
        in_specs=[pl.BlockSpec((_RB, RP128, CODEBOOK_DIM),
                               lambda i: (i, 0, 0))],
        out_specs=pl.BlockSpec((_RB, 128), lambda i: (i, 0)),
        out_shape=jax.ShapeDtypeStruct((V // RP128, 128), jnp.float32),
        compiler_params=pltpu.CompilerParams(
            dimension_semantics=("arbitrary",),
        ),
    )(table3d)


_NC, _NS = 2, 16
_NW = _NC * _NS
_B_PER_W = T // _NW
_CHUNK = 128
_NCHUNK = _B_PER_W // _CHUNK


@functools.cache
def _gather_sc():
    @functools.partial(
        pl.kernel,
        mesh=plsc.VectorSubcoreMesh(core_axis_name="c", subcore_axis_name="s"),
        compiler_params=pltpu.CompilerParams(use_tc_tiling_on_sc=False),
        out_type=jax.ShapeDtypeStruct((T, CODEBOOK_DIM), jnp.float32),
        scratch_types=[
            pltpu.VMEM((_NCHUNK, _CHUNK), jnp.int32),
            pltpu.VMEM((_B_PER_W, CODEBOOK_DIM), jnp.float32),
            pltpu.SemaphoreType.DMA,
        ],
    )
    def gather(ids_hbm, table_hbm, out_hbm, idx_v, rows_v, sem):
        wid = lax.axis_index("s") * _NC + lax.axis_index("c")
        pltpu.sync_copy(ids_hbm.at[pl.ds(wid * _NCHUNK, _NCHUNK)], idx_v)
        copies = [
            pltpu.async_copy(
                table_hbm.at[idx_v.at[j]],
                rows_v.at[pl.ds(j * _CHUNK, _CHUNK)],
                sem,
            )
            for j in range(_NCHUNK)
        ]
        for cp in copies:
            cp.wait()
        pltpu.sync_copy(rows_v, out_hbm.at[pl.ds(wid * _B_PER_W, _B_PER_W)])

    return gather


_TT = 1024
_NT = T // _TT


def _mlp_body(e_ref, id_ref, w1_ref, w2_ref, b2_ref, o_ref):
    e = e_ref[...]
    ids = id_ref[...]
    c = lax.shift_right_logical(ids, CODEBOOK_BITS)

    oh4 = (c == lax.broadcasted_iota(jnp.int32, (1, NUM_CODEBOOKS), 1))
    oh4b = oh4.astype(jnp.bfloat16)
    eb = e.astype(jnp.bfloat16)

    placed = jnp.concatenate(
        [eb * oh4b[:, i:i + 1] for i in range(NUM_CODEBOOKS)] + [oh4b],
        axis=1,
    )
    h = lax.dot_general(
        placed, w1_ref[...], (((1,), (0,)), ((), ())),
        preferred_element_type=jnp.float32,
    )

    g = 0.5 * h * (1.0 + lax.erf(h * 0.7071067811865476))
    gb = g.astype(jnp.bfloat16)

    gwide = jnp.concatenate(
        [gb * oh4b[:, i:i + 1] for i in range(NUM_CODEBOOKS)],
        axis=1,
    )
    bias2 = lax.dot_general(
        oh4b, b2_ref[...], (((1,), (0,)), ((), ())),
        preferred_element_type=jnp.float32,
    )
    o_ref[...] = bias2 + lax.dot_general(
        gwide, w2_ref[...], (((1,), (0,)), ((), ())),
        preferred_element_type=jnp.float32,
    )


def _mlp_tc(embeds, ids_col, w1ext, w2s, b2b):
    return pl.pallas_call(
        _mlp_body,
        grid=(_NT,),
        in_specs=[
            pl.BlockSpec((_TT, CODEBOOK_DIM), lambda i: (i, 0)),
            pl.BlockSpec((_TT, 1), lambda i: (i, 0)),
            pl.BlockSpec((NUM_CODEBOOKS * CODEBOOK_DIM + NUM_CODEBOOKS,
                          HIDDEN_SIZE), lambda i: (0, 0)),
            pl.BlockSpec((NUM_CODEBOOKS * HIDDEN_SIZE, HIDDEN_SIZE),
                         lambda i: (0, 0)),
            pl.BlockSpec((NUM_CODEBOOKS, HIDDEN_SIZE), lambda i: (0, 0)),
        ],
        out_specs=pl.BlockSpec((_TT, HIDDEN_SIZE), lambda i: (i, 0)),
        out_shape=jax.ShapeDtypeStruct((T, HIDDEN_SIZE), jnp.float32),
        compiler_params=pltpu.CompilerParams(
            dimension_semantics=("arbitrary",),
        ),
    )(embeds, ids_col, w1ext, w2s, b2b)


def kernel(codec_input_ids, table, W1, b1, W2, b2):
    ids = codec_input_ids.reshape(-1).astype(jnp.int32)
    ids128 = lax.rem(ids.reshape(_NW * _NCHUNK, _CHUNK),
                     jnp.int32(1 << 30))
    table128 = lax.rem(table.reshape(V // RP128, 128), jnp.float32(1e30))
    embeds = _gather_sc()(ids128, table128.reshape(V, CODEBOOK_DIM))
    w1ext = jnp.concatenate(
        [W1.reshape(NUM_CODEBOOKS * CODEBOOK_DIM, HIDDEN_SIZE), b1], axis=0
    ).astype(jnp.bfloat16)
    w2s = W2.reshape(NUM_CODEBOOKS * HIDDEN_SIZE,
                     HIDDEN_SIZE).astype(jnp.bfloat16)
    out = _mlp_tc(embeds, ids.reshape(T, 1), w1ext, w2s,
                  b2.astype(jnp.bfloat16))
    return out.reshape(B, S, HIDDEN_SIZE)


# speedup vs baseline: 1.5617x; 1.5617x over previous
import functools

import jax
import jax.numpy as jnp
from jax import lax
from jax.experimental import pallas as pl
from jax.experimental.pallas import tpu as pltpu
from jax.experimental.pallas import tpu_sc as plsc

NUM_CODEBOOKS = 4
CODEBOOK_BITS = 17
CODEBOOK_DIM = 16
HIDDEN_SIZE = 768
B, S = 4, 4096
T = B * S
V = NUM_CODEBOOKS * (1 << CODEBOOK_BITS)
RP128 = 128 // CODEBOOK_DIM

_RB = 2048
_NRB = V // RP128 // _RB


def _repack_body(in_ref, out_ref):
    for k in range(RP128):
        out_ref[:, k * CODEBOOK_DIM:(k + 1) * CODEBOOK_DIM] = in_ref[:, k, :]


def _repack_tc(table3d):
    return pl.pallas_call(
        _repack_body,
        grid=(_NRB,),
        in_specs=[pl.BlockSpec((_RB, RP128, CODEBOOK_DIM),
                               lambda i: (i, 0, 0))],
        out_specs=pl.BlockSpec((_RB, 128), lambda i: (i, 0)),
        out_shape=jax.ShapeDtypeStruct((V // RP128, 128), jnp.float32),
        compiler_params=pltpu.CompilerParams(
            dimension_semantics=("arbitrary",),
        ),
    )(table3d)


_NC, _NS = 2, 16
_NW = _NC * _NS
_B_PER_W = T // _NW
_CHUNK = 128
_NCHUNK = _B_PER_W // _CHUNK


@functools.cache
def _gather_sc():
    @functools.partial(
        pl.kernel,
        mesh=plsc.VectorSubcoreMesh(core_axis_name="c", subcore_axis_name="s"),
        compiler_params=pltpu.CompilerParams(use_tc_tiling_on_sc=False),
        out_type=jax.ShapeDtypeStruct((T, CODEBOOK_DIM), jnp.float32),
        scratch_types=[
            pltpu.VMEM((_NCHUNK, _CHUNK), jnp.int32),
            pltpu.VMEM((_B_PER_W, CODEBOOK_DIM), jnp.float32),
            pltpu.SemaphoreType.DMA,
        ],
    )
    def gather(ids_hbm, table_hbm, out_hbm, idx_v, rows_v, sem):
        wid = lax.axis_index("s") * _NC + lax.axis_index("c")
        pltpu.sync_copy(ids_hbm.at[pl.ds(wid * _NCHUNK, _NCHUNK)], idx_v)
        copies = [
            pltpu.async_copy(
                table_hbm.at[idx_v.at[j]],
                rows_v.at[pl.ds(j * _CHUNK, _CHUNK)],
                sem,
            )
            for j in range(_NCHUNK)
        ]
        for cp in copies:
            cp.wait()
        pltpu.sync_copy(rows_v, out_hbm.at[pl.ds(wid * _B_PER_W, _B_PER_W)])

    return gather


_TT = 1024
_NT = T // _TT


def _mlp_body(e_ref, id_ref, w1_ref, w2_ref, b2_ref, o_ref):
    e = e_ref[...]
    ids = id_ref[...]
    c = lax.shift_right_logical(ids, CODEBOOK_BITS)

    oh4 = (c == lax.broadcasted_iota(jnp.int32, (1, NUM_CODEBOOKS), 1))
    oh4b = oh4.astype(jnp.bfloat16)
    eb = e.astype(jnp.bfloat16)

    placed = jnp.concatenate(
        [eb * oh4b[:, i:i + 1] for i in range(NUM_CODEBOOKS)] + [oh4b],
        axis=1,
    )
    h = lax.dot_general(
        placed, w1_ref[...], (((1,), (0,)), ((), ())),
        preferred_element_type=jnp.float32,
    )

    g = 0.5 * h * (1.0 + lax.erf(h * 0.7071067811865476))
    gb = g.astype(jnp.bfloat16)

    gwide = jnp.concatenate(
        [gb * oh4b[:, i:i + 1] for i in range(NUM_CODEBOOKS)],
        axis=1,
    )
    bias2 = lax.dot_general(
        oh4b, b2_ref[...], (((1,), (0,)), ((), ())),
        preferred_element_type=jnp.float32,
    )
    o_ref[...] = bias2 + lax.dot_general(
        gwide, w2_ref[...], (((1,), (0,)), ((), ())),
        preferred_element_type=jnp.float32,
    )


def _mlp_tc(embeds, ids_col, w1ext, w2s, b2b):
    return pl.pallas_call(
        _mlp_body,
        grid=(_NT,),
        in_specs=[
            pl.BlockSpec((_TT, CODEBOOK_DIM), lambda i: (i, 0)),
            pl.BlockSpec((_TT, 1), lambda i: (i, 0)),
            pl.BlockSpec((NUM_CODEBOOKS * CODEBOOK_DIM + NUM_CODEBOOKS,
                          HIDDEN_SIZE), lambda i: (0, 0)),
            pl.BlockSpec((NUM_CODEBOOKS * HIDDEN_SIZE, HIDDEN_SIZE),
                         lambda i: (0, 0)),
            pl.BlockSpec((NUM_CODEBOOKS, HIDDEN_SIZE), lambda i: (0, 0)),
        ],
        out_specs=pl.BlockSpec((_TT, HIDDEN_SIZE), lambda i: (i, 0)),
        out_shape=jax.ShapeDtypeStruct((T, HIDDEN_SIZE), jnp.float32),
        compiler_params=pltpu.CompilerParams(
            dimension_semantics=("arbitrary",),
        ),
    )(embeds, ids_col, w1ext, w2s, b2b)


def kernel(codec_input_ids, table, W1, b1, W2, b2):
    ids = codec_input_ids.reshape(-1).astype(jnp.int32)
    ids128 = lax.rem(ids.reshape(_NW * _NCHUNK, _CHUNK),
                     jnp.int32(1 << 30))
    table128 = _repack_tc(table.reshape(-1, RP128, CODEBOOK_DIM))
    embeds = _gather_sc()(ids128, table128.reshape(V, CODEBOOK_DIM))
    w1ext = jnp.concatenate(
        [W1.reshape(NUM_CODEBOOKS * CODEBOOK_DIM, HIDDEN_SIZE), b1], axis=0
    ).astype(jnp.bfloat16)
    w2s = W2.reshape(NUM_CODEBOOKS * HIDDEN_SIZE,
                     HIDDEN_SIZE).astype(jnp.bfloat16)
    out = _mlp_tc(embeds, ids.reshape(T, 1), w1ext, w2s,
                  b2.astype(jnp.bfloat16))
    return out.reshape(B, S, HIDDEN_SIZE)
